# baseline (speedup 1.0000x reference)
import jax
import jax.numpy as jnp
from jax.experimental import pallas as pl
from jax.experimental.pallas import tpu as pltpu

EPS = 1e-5
VMEM_LIMIT = 32 * 1024 * 1024


def _build_colpat(cp_ref, img):
    H, W, C = img.shape
    z = jnp.zeros((1, W, 3 * C), jnp.float32)
    cp_ref[0:1] = z
    cp_ref[H + 1:H + 2] = z
    zc = jnp.zeros((H, 1, C), jnp.float32)
    cp_ref[1:H + 1, 0:1, 0:C] = zc
    cp_ref[1:H + 1, 1:W, 0:C] = img[:, 0:W - 1, :]
    cp_ref[1:H + 1, :, C:2 * C] = img
    cp_ref[1:H + 1, 0:W - 1, 2 * C:3 * C] = img[:, 1:W, :]
    cp_ref[1:H + 1, W - 1:W, 2 * C:3 * C] = zc


def _prep_kernel(x_ref, feat_ref, fcw_ref, fcb_ref, s1_ref, sh1_ref, aff2_ref):
    N, H, W, C = x_ref.shape
    M = N * H * W
    x = x_ref[...].reshape(M, C)
    aff = jnp.dot(feat_ref[...], fcw_ref[...],
                  preferred_element_type=jnp.float32) + fcb_ref[...]
    mean1 = jnp.sum(x, axis=0, keepdims=True) / M
    ex2 = jnp.sum(x * x, axis=0, keepdims=True) / M
    inv1 = jax.lax.rsqrt(ex2 - mean1 * mean1 + EPS)
    wv1 = aff[:, 0 * C:1 * C]
    bv1 = aff[:, 1 * C:2 * C]
    s1_ref[...] = (wv1 * inv1).reshape(N, 1, C)
    sh1_ref[...] = (bv1 - wv1 * mean1 * inv1).reshape(N, 1, C)
    aff2_ref[...] = aff[:, 2 * C:4 * C].reshape(N, 1, 2 * C)


def _conv1_kernel(x_ref, s1_ref, sh1_ref, w1p_ref, b1_ref, t_ref, part_ref,
                  cp_ref):
    _, H, W, C = x_ref.shape
    M = H * W
    x = x_ref[0].reshape(M, C)
    h = jnp.maximum(s1_ref[0] * x + sh1_ref[0], 0.0)
    _build_colpat(cp_ref, h.reshape(H, W, C))
    flat = cp_ref[...].reshape((H + 2) * W, 3 * C)
    s = None
    s2 = None
    for a in range(2):
        for b in range(2):
            acc = None
            for u in range(2):
                lhs = flat[(a + u) * W:(a + u) * W + M, b * C:(b + 2) * C]
                d = jnp.dot(lhs, w1p_ref[(a * 2 + b) * 2 + u],
                            preferred_element_type=jnp.float32)
                acc = d if acc is None else acc + d
            conv = acc + b1_ref[...]
            t_ref[0, a * 2 + b] = conv.astype(jnp.bfloat16).reshape(H, W, C)
            ps = jnp.sum(conv, axis=0, keepdims=True)
            ps2 = jnp.sum(conv * conv, axis=0, keepdims=True)
            s = ps if s is None else s + ps
            s2 = ps2 if s2 is None else s2 + ps2
    part_ref[0, 0:1, :] = s
    part_ref[0, 1:2, :] = s2


def _conv2_kernel(t_ref, part_ref, aff2_ref, x_ref, w3_ref, b3_ref, w2_ref,
                  b2_ref, out_ref, cpz_ref):
    _, _, H, W, C = t_ref.shape
    M = H * W
    N = part_ref.shape[0]
    Cout = w3_ref.shape[1]
    M2 = N * 4 * H * W
    mean2 = jnp.sum(part_ref[:, 0, :], axis=0, keepdims=True) / M2
    ex2 = jnp.sum(part_ref[:, 1, :], axis=0, keepdims=True) / M2
    inv2 = jax.lax.rsqrt(ex2 - mean2 * mean2 + EPS)
    wv2 = aff2_ref[0, :, 0:C]
    bv2 = aff2_ref[0, :, C:2 * C]
    sc = wv2 * inv2
    sh = bv2 - wv2 * mean2 * inv2

    zp = []
    for ph in range(4):
        tp = t_ref[0, ph].reshape(M, C).astype(jnp.float32)
        zp.append(jnp.maximum(sc * tp + sh, 0.0).reshape(H, W, C))

    zcol = jnp.zeros((H, 1, C), jnp.float32)
    zrow = jnp.zeros((1, W, 3 * C), jnp.float32)
    for a in range(2):
        z0 = zp[a * 2 + 0]
        z1 = zp[a * 2 + 1]
        for bp in range(2):
            i = a * 2 + bp
            cpz_ref[i, 0:1] = zrow
            cpz_ref[i, H + 1:H + 2] = zrow
            if bp == 0:
                cpz_ref[i, 1:H + 1, 0:1, 0:C] = zcol
                cpz_ref[i, 1:H + 1, 1:W, 0:C] = z1[:, 0:W - 1, :]
                cpz_ref[i, 1:H + 1, :, C:2 * C] = z0
                cpz_ref[i, 1:H + 1, :, 2 * C:3 * C] = z1
            else:
                cpz_ref[i, 1:H + 1, :, 0:C] = z0
                cpz_ref[i, 1:H + 1, :, C:2 * C] = z1
                cpz_ref[i, 1:H + 1, 0:W - 1, 2 * C:3 * C] = z0[:, 1:W, :]
                cpz_ref[i, 1:H + 1, W - 1:W, 2 * C:3 * C] = zcol

    xs = x_ref[0].reshape(M, C)
    skip = jnp.dot(xs, w3_ref[...],
                   preferred_element_type=jnp.float32) + b3_ref[...]

    flats = [cpz_ref[i].reshape((H + 2) * W, 3 * C) for i in range(4)]
    for ap in range(2):
        for bp in range(2):
            acc = None
            for dy in range(3):
                m = ap - 1 + dy
                a_src = m % 2
                u = m // 2
                lhs = flats[a_src * 2 + bp][(1 + u) * W:(1 + u) * W + M]
                d = jnp.dot(lhs, w2_ref[dy],
                            preferred_element_type=jnp.float32)
                acc = d if acc is None else acc + d
            outp = acc + b2_ref[...] + skip
            out_ref[0, ap, bp] = outp.reshape(H, W, Cout)


def kernel(x, feat, w1, b1, w2, b2, w3, b3, fcw1_w, fcw1_b, fcb1_w, fcb1_b,
           fcw2_w, fcw2_b, fcb2_w, fcb2_b):
    N, Cin, H, W = x.shape
    Cout = w2.shape[0]
    f32, bf16 = jnp.float32, jnp.bfloat16

    x_lo = jnp.transpose(x, (0, 2, 3, 1)).astype(f32)

    whwio = jnp.transpose(w1, (2, 3, 1, 0))
    rowc = [[whwio[0], whwio[1] + whwio[2]],
            [whwio[0] + whwio[1], whwio[2]]]
    blocks = []
    for a in range(2):
        for b in range(2):
            for u in range(2):
                r = rowc[a][u]
                colc = [[r[0], r[1] + r[2]], [r[0] + r[1], r[2]]]
                blocks.append(jnp.concatenate([colc[b][0], colc[b][1]],
                                              axis=0))
    w1p = jnp.stack(blocks).astype(f32)

    w2r = jnp.transpose(w2, (2, 3, 1, 0)).reshape(3, 3 * Cin, Cout).astype(f32)
    w3m = jnp.transpose(w3[:, :, 0, 0], (1, 0)).astype(f32)
    b1m = b1.reshape(1, Cin).astype(f32)
    b2m = b2.reshape(1, Cout).astype(f32)
    b3m = b3.reshape(1, Cout).astype(f32)
    fc_w = jnp.concatenate([fcw1_w, fcb1_w, fcw2_w, fcb2_w],
                           axis=1).astype(f32)
    fc_b = jnp.concatenate([fcw1_b, fcb1_b, fcw2_b, fcb2_b]).reshape(1, 4 * Cin)

    vmem = pl.BlockSpec(memory_space=pltpu.MemorySpace.VMEM)

    s1, sh1, aff2 = pl.pallas_call(
        _prep_kernel,
        out_shape=(jax.ShapeDtypeStruct((N, 1, Cin), f32),
                   jax.ShapeDtypeStruct((N, 1, Cin), f32),
                   jax.ShapeDtypeStruct((N, 1, 2 * Cin), f32)),
        in_specs=[vmem] * 4,
        out_specs=(vmem, vmem, vmem),
        compiler_params=pltpu.CompilerParams(vmem_limit_bytes=VMEM_LIMIT),
    )(x_lo, feat.astype(f32), fc_w, fc_b)

    t, part = pl.pallas_call(
        _conv1_kernel,
        out_shape=(jax.ShapeDtypeStruct((N, 4, H, W, Cin), bf16),
                   jax.ShapeDtypeStruct((N, 2, Cin), f32)),
        grid=(N,),
        in_specs=[pl.BlockSpec((1, H, W, Cin), lambda n: (n, 0, 0, 0)),
                  pl.BlockSpec((1, 1, Cin), lambda n: (n, 0, 0)),
                  pl.BlockSpec((1, 1, Cin), lambda n: (n, 0, 0)),
                  pl.BlockSpec((8, 2 * Cin, Cin), lambda n: (0, 0, 0)),
                  pl.BlockSpec((1, Cin), lambda n: (0, 0))],
        out_specs=(pl.BlockSpec((1, 4, H, W, Cin), lambda n: (n, 0, 0, 0, 0)),
                   pl.BlockSpec((1, 2, Cin), lambda n: (n, 0, 0))),
        scratch_shapes=[pltpu.VMEM((H + 2, W, 3 * Cin), f32)],
        compiler_params=pltpu.CompilerParams(
            dimension_semantics=("parallel",), vmem_limit_bytes=VMEM_LIMIT),
    )(x_lo, s1, sh1, w1p, b1m)

    out_ph = pl.pallas_call(
        _conv2_kernel,
        out_shape=jax.ShapeDtypeStruct((N, 2, 2, H, W, Cout), f32),
        grid=(N,),
        in_specs=[pl.BlockSpec((1, 4, H, W, Cin), lambda n: (n, 0, 0, 0, 0)),
                  pl.BlockSpec((N, 2, Cin), lambda n: (0, 0, 0)),
                  pl.BlockSpec((1, 1, 2 * Cin), lambda n: (n, 0, 0)),
                  pl.BlockSpec((1, H, W, Cin), lambda n: (n, 0, 0, 0)),
                  pl.BlockSpec((Cin, Cout), lambda n: (0, 0)),
                  pl.BlockSpec((1, Cout), lambda n: (0, 0)),
                  pl.BlockSpec((3, 3 * Cin, Cout), lambda n: (0, 0, 0)),
                  pl.BlockSpec((1, Cout), lambda n: (0, 0))],
        out_specs=pl.BlockSpec((1, 2, 2, H, W, Cout),
                               lambda n: (n, 0, 0, 0, 0, 0)),
        scratch_shapes=[pltpu.VMEM((4, H + 2, W, 3 * Cin), f32)],
        compiler_params=pltpu.CompilerParams(
            dimension_semantics=("parallel",), vmem_limit_bytes=VMEM_LIMIT),
    )(t, part, aff2, x_lo, w3m, b3m, w2r, b2m)

    out = jnp.transpose(out_ph, (0, 5, 3, 1, 4, 2))
    return out.reshape(N, Cout, 2 * H, 2 * W)


# speedup vs baseline: 1.6966x; 1.6966x over previous
import jax
import jax.numpy as jnp
from jax.experimental import pallas as pl
from jax.experimental.pallas import tpu as pltpu

EPS = 1e-5
VMEM_LIMIT = 32 * 1024 * 1024


def _build_colpat(cp_ref, img):
    H, W, C = img.shape
    z = jnp.zeros((1, W, 3 * C), jnp.float32)
    cp_ref[0:1] = z
    cp_ref[H + 1:H + 2] = z
    zc = jnp.zeros((H, 1, C), jnp.float32)
    cp_ref[1:H + 1, 0:1, 0:C] = zc
    cp_ref[1:H + 1, 1:W, 0:C] = img[:, 0:W - 1, :]
    cp_ref[1:H + 1, :, C:2 * C] = img
    cp_ref[1:H + 1, 0:W - 1, 2 * C:3 * C] = img[:, 1:W, :]
    cp_ref[1:H + 1, W - 1:W, 2 * C:3 * C] = zc


def _prep_kernel(x_ref, feat_ref, fcw_ref, fcb_ref, s1_ref, sh1_ref, aff2_ref):
    N, H, W, C = x_ref.shape
    M = N * H * W
    x = x_ref[...].reshape(M, C)
    aff = jnp.dot(feat_ref[...], fcw_ref[...],
                  preferred_element_type=jnp.float32) + fcb_ref[...]
    mean1 = jnp.sum(x, axis=0, keepdims=True) / M
    ex2 = jnp.sum(x * x, axis=0, keepdims=True) / M
    inv1 = jax.lax.rsqrt(ex2 - mean1 * mean1 + EPS)
    wv1 = aff[:, 0 * C:1 * C]
    bv1 = aff[:, 1 * C:2 * C]
    s1_ref[...] = (wv1 * inv1).reshape(N, 1, C)
    sh1_ref[...] = (bv1 - wv1 * mean1 * inv1).reshape(N, 1, C)
    aff2_ref[...] = aff[:, 2 * C:4 * C].reshape(N, 1, 2 * C)


def _conv1_kernel(x_ref, s1_ref, sh1_ref, w1p_ref, b1_ref, t_ref, part_ref,
                  cp_ref):
    _, H, W, C = x_ref.shape
    M = H * W
    x = x_ref[0].reshape(M, C)
    h = jnp.maximum(s1_ref[0] * x + sh1_ref[0], 0.0)
    _build_colpat(cp_ref, h.reshape(H, W, C))
    flat = cp_ref[...].reshape((H + 2) * W, 3 * C)
    s = None
    s2 = None
    for a in range(2):
        for b in range(2):
            acc = None
            for u in range(2):
                lhs = flat[(a + u) * W:(a + u) * W + M, b * C:(b + 2) * C]
                d = jnp.dot(lhs, w1p_ref[(a * 2 + b) * 2 + u],
                            preferred_element_type=jnp.float32)
                acc = d if acc is None else acc + d
            conv = acc + b1_ref[...]
            t_ref[0, a * 2 + b] = conv.astype(jnp.bfloat16).reshape(H, W, C)
            ps = jnp.sum(conv, axis=0, keepdims=True)
            ps2 = jnp.sum(conv * conv, axis=0, keepdims=True)
            s = ps if s is None else s + ps
            s2 = ps2 if s2 is None else s2 + ps2
    part_ref[0, 0:1, :] = s
    part_ref[0, 1:2, :] = s2


def _conv2_kernel(t_ref, part_ref, aff2_ref, x_ref, w3_ref, b3_ref, w2_ref,
                  b2_ref, out_ref, cpz_ref):
    _, _, H, W, C = t_ref.shape
    M = H * W
    N = part_ref.shape[0]
    Cout = w3_ref.shape[1]
    M2 = N * 4 * H * W
    mean2 = jnp.sum(part_ref[:, 0, :], axis=0, keepdims=True) / M2
    ex2 = jnp.sum(part_ref[:, 1, :], axis=0, keepdims=True) / M2
    inv2 = jax.lax.rsqrt(ex2 - mean2 * mean2 + EPS)
    wv2 = aff2_ref[0, :, 0:C]
    bv2 = aff2_ref[0, :, C:2 * C]
    sc = wv2 * inv2
    sh = bv2 - wv2 * mean2 * inv2

    zp = []
    for ph in range(4):
        tp = t_ref[0, ph].reshape(M, C).astype(jnp.float32)
        zp.append(jnp.maximum(sc * tp + sh, 0.0).reshape(H, W, C))

    zcol = jnp.zeros((H, 1, C), jnp.float32)
    zrow = jnp.zeros((1, W, 3 * C), jnp.float32)
    for a in range(2):
        z0 = zp[a * 2 + 0]
        z1 = zp[a * 2 + 1]
        for bp in range(2):
            i = a * 2 + bp
            cpz_ref[i, 0:1] = zrow
            cpz_ref[i, H + 1:H + 2] = zrow
            if bp == 0:
                cpz_ref[i, 1:H + 1, 0:1, 0:C] = zcol
                cpz_ref[i, 1:H + 1, 1:W, 0:C] = z1[:, 0:W - 1, :]
                cpz_ref[i, 1:H + 1, :, C:2 * C] = z0
                cpz_ref[i, 1:H + 1, :, 2 * C:3 * C] = z1
            else:
                cpz_ref[i, 1:H + 1, :, 0:C] = z0
                cpz_ref[i, 1:H + 1, :, C:2 * C] = z1
                cpz_ref[i, 1:H + 1, 0:W - 1, 2 * C:3 * C] = z0[:, 1:W, :]
                cpz_ref[i, 1:H + 1, W - 1:W, 2 * C:3 * C] = zcol

    xs = x_ref[0].reshape(M, C)
    skip = jnp.dot(xs, w3_ref[...],
                   preferred_element_type=jnp.float32) + b3_ref[...]

    flats = [cpz_ref[i].reshape((H + 2) * W, 3 * C) for i in range(4)]
    rows = []
    for ap in range(2):
        cols = []
        for bp in range(2):
            acc = None
            for dy in range(3):
                m = ap - 1 + dy
                a_src = m % 2
                u = m // 2
                lhs = flats[a_src * 2 + bp][(1 + u) * W:(1 + u) * W + M]
                d = jnp.dot(lhs, w2_ref[dy],
                            preferred_element_type=jnp.float32)
                acc = d if acc is None else acc + d
            cols.append((acc + b2_ref[...] + skip).reshape(H, W, Cout))
        rows.append(jnp.stack(cols, axis=2).reshape(H, 2 * W, Cout))
    out_ref[0] = jnp.stack(rows, axis=1).reshape(2 * H, 2 * W, Cout)


def kernel(x, feat, w1, b1, w2, b2, w3, b3, fcw1_w, fcw1_b, fcb1_w, fcb1_b,
           fcw2_w, fcw2_b, fcb2_w, fcb2_b):
    N, Cin, H, W = x.shape
    Cout = w2.shape[0]
    f32, bf16 = jnp.float32, jnp.bfloat16

    x_lo = jnp.transpose(x, (0, 2, 3, 1)).astype(f32)

    whwio = jnp.transpose(w1, (2, 3, 1, 0))
    rowc = [[whwio[0], whwio[1] + whwio[2]],
            [whwio[0] + whwio[1], whwio[2]]]
    blocks = []
    for a in range(2):
        for b in range(2):
            for u in range(2):
                r = rowc[a][u]
                colc = [[r[0], r[1] + r[2]], [r[0] + r[1], r[2]]]
                blocks.append(jnp.concatenate([colc[b][0], colc[b][1]],
                                              axis=0))
    w1p = jnp.stack(blocks).astype(f32)

    w2r = jnp.transpose(w2, (2, 3, 1, 0)).reshape(3, 3 * Cin, Cout).astype(f32)
    w3m = jnp.transpose(w3[:, :, 0, 0], (1, 0)).astype(f32)
    b1m = b1.reshape(1, Cin).astype(f32)
    b2m = b2.reshape(1, Cout).astype(f32)
    b3m = b3.reshape(1, Cout).astype(f32)
    fc_w = jnp.concatenate([fcw1_w, fcb1_w, fcw2_w, fcb2_w],
                           axis=1).astype(f32)
    fc_b = jnp.concatenate([fcw1_b, fcb1_b, fcw2_b, fcb2_b]).reshape(1, 4 * Cin)

    vmem = pl.BlockSpec(memory_space=pltpu.MemorySpace.VMEM)

    s1, sh1, aff2 = pl.pallas_call(
        _prep_kernel,
        out_shape=(jax.ShapeDtypeStruct((N, 1, Cin), f32),
                   jax.ShapeDtypeStruct((N, 1, Cin), f32),
                   jax.ShapeDtypeStruct((N, 1, 2 * Cin), f32)),
        in_specs=[vmem] * 4,
        out_specs=(vmem, vmem, vmem),
        compiler_params=pltpu.CompilerParams(vmem_limit_bytes=VMEM_LIMIT),
    )(x_lo, feat.astype(f32), fc_w, fc_b)

    t, part = pl.pallas_call(
        _conv1_kernel,
        out_shape=(jax.ShapeDtypeStruct((N, 4, H, W, Cin), bf16),
                   jax.ShapeDtypeStruct((N, 2, Cin), f32)),
        grid=(N,),
        in_specs=[pl.BlockSpec((1, H, W, Cin), lambda n: (n, 0, 0, 0)),
                  pl.BlockSpec((1, 1, Cin), lambda n: (n, 0, 0)),
                  pl.BlockSpec((1, 1, Cin), lambda n: (n, 0, 0)),
                  pl.BlockSpec((8, 2 * Cin, Cin), lambda n: (0, 0, 0)),
                  pl.BlockSpec((1, Cin), lambda n: (0, 0))],
        out_specs=(pl.BlockSpec((1, 4, H, W, Cin), lambda n: (n, 0, 0, 0, 0)),
                   pl.BlockSpec((1, 2, Cin), lambda n: (n, 0, 0))),
        scratch_shapes=[pltpu.VMEM((H + 2, W, 3 * Cin), f32)],
        compiler_params=pltpu.CompilerParams(
            dimension_semantics=("parallel",), vmem_limit_bytes=VMEM_LIMIT),
    )(x_lo, s1, sh1, w1p, b1m)

    out_nhwc = pl.pallas_call(
        _conv2_kernel,
        out_shape=jax.ShapeDtypeStruct((N, 2 * H, 2 * W, Cout), f32),
        grid=(N,),
        in_specs=[pl.BlockSpec((1, 4, H, W, Cin), lambda n: (n, 0, 0, 0, 0)),
                  pl.BlockSpec((N, 2, Cin), lambda n: (0, 0, 0)),
                  pl.BlockSpec((1, 1, 2 * Cin), lambda n: (n, 0, 0)),
                  pl.BlockSpec((1, H, W, Cin), lambda n: (n, 0, 0, 0)),
                  pl.BlockSpec((Cin, Cout), lambda n: (0, 0)),
                  pl.BlockSpec((1, Cout), lambda n: (0, 0)),
                  pl.BlockSpec((3, 3 * Cin, Cout), lambda n: (0, 0, 0)),
                  pl.BlockSpec((1, Cout), lambda n: (0, 0))],
        out_specs=pl.BlockSpec((1, 2 * H, 2 * W, Cout),
                               lambda n: (n, 0, 0, 0)),
        scratch_shapes=[pltpu.VMEM((4, H + 2, W, 3 * Cin), f32)],
        compiler_params=pltpu.CompilerParams(
            dimension_semantics=("parallel",), vmem_limit_bytes=VMEM_LIMIT),
    )(t, part, aff2, x_lo, w3m, b3m, w2r, b2m)

    return jnp.transpose(out_nhwc, (0, 3, 1, 2))


# speedup vs baseline: 1.9272x; 1.1360x over previous
import jax
import jax.numpy as jnp
from jax.experimental import pallas as pl
from jax.experimental.pallas import tpu as pltpu

EPS = 1e-5
VMEM_LIMIT = 32 * 1024 * 1024


def _build_colpat(cp_ref, img):
    H, W, C = img.shape
    z = jnp.zeros((1, W, 3 * C), jnp.float32)
    cp_ref[0:1] = z
    cp_ref[H + 1:H + 2] = z
    zc = jnp.zeros((H, 1, C), jnp.float32)
    cp_ref[1:H + 1, 0:1, 0:C] = zc
    cp_ref[1:H + 1, 1:W, 0:C] = img[:, 0:W - 1, :]
    cp_ref[1:H + 1, :, C:2 * C] = img
    cp_ref[1:H + 1, 0:W - 1, 2 * C:3 * C] = img[:, 1:W, :]
    cp_ref[1:H + 1, W - 1:W, 2 * C:3 * C] = zc


def _prep_kernel(x_ref, feat_ref, fcw_ref, fcb_ref, s1_ref, sh1_ref, aff2_ref):
    N, H, W, C = x_ref.shape
    M = N * H * W
    x = x_ref[...].reshape(M, C)
    aff = jnp.dot(feat_ref[...], fcw_ref[...],
                  preferred_element_type=jnp.float32) + fcb_ref[...]
    mean1 = jnp.sum(x, axis=0, keepdims=True) / M
    ex2 = jnp.sum(x * x, axis=0, keepdims=True) / M
    inv1 = jax.lax.rsqrt(ex2 - mean1 * mean1 + EPS)
    wv1 = aff[:, 0 * C:1 * C]
    bv1 = aff[:, 1 * C:2 * C]
    s1_ref[...] = (wv1 * inv1).reshape(N, 1, C)
    sh1_ref[...] = (bv1 - wv1 * mean1 * inv1).reshape(N, 1, C)
    aff2_ref[...] = aff[:, 2 * C:4 * C].reshape(N, 1, 2 * C)


def _conv1_kernel(x_ref, s1_ref, sh1_ref, w1p_ref, b1_ref, t_ref, part_ref,
                  cp_ref):
    _, H, W, C = x_ref.shape
    M = H * W
    x = x_ref[0].reshape(M, C)
    h = jnp.maximum(s1_ref[0] * x + sh1_ref[0], 0.0)
    _build_colpat(cp_ref, h.reshape(H, W, C))
    flat = cp_ref[...].reshape((H + 2) * W, 3 * C)
    s = None
    s2 = None
    for a in range(2):
        for b in range(2):
            acc = None
            for u in range(2):
                lhs = flat[(a + u) * W:(a + u) * W + M, b * C:(b + 2) * C]
                d = jnp.dot(lhs, w1p_ref[(a * 2 + b) * 2 + u],
                            preferred_element_type=jnp.float32)
                acc = d if acc is None else acc + d
            conv = acc + b1_ref[...]
            t_ref[0, a * 2 + b] = conv.astype(jnp.bfloat16).reshape(H, W, C)
            ps = jnp.sum(conv, axis=0, keepdims=True)
            ps2 = jnp.sum(conv * conv, axis=0, keepdims=True)
            s = ps if s is None else s + ps
            s2 = ps2 if s2 is None else s2 + ps2
    part_ref[0, 0:1, :] = s
    part_ref[0, 1:2, :] = s2


def _conv2_kernel(t_ref, part_ref, aff2_ref, x_ref, w3_ref, b3_ref, w2_ref,
                  b2_ref, out_ref, cpz_ref):
    _, _, H, W, C = t_ref.shape
    M = H * W
    N = part_ref.shape[0]
    Cout = w3_ref.shape[1]
    M2 = N * 4 * H * W
    mean2 = jnp.sum(part_ref[:, 0, :], axis=0, keepdims=True) / M2
    ex2 = jnp.sum(part_ref[:, 1, :], axis=0, keepdims=True) / M2
    inv2 = jax.lax.rsqrt(ex2 - mean2 * mean2 + EPS)
    wv2 = aff2_ref[0, :, 0:C]
    bv2 = aff2_ref[0, :, C:2 * C]
    sc = wv2 * inv2
    sh = bv2 - wv2 * mean2 * inv2

    zp = []
    for ph in range(4):
        tp = t_ref[0, ph].reshape(M, C).astype(jnp.float32)
        zp.append(jnp.maximum(sc * tp + sh, 0.0).reshape(H, W, C))

    zcol = jnp.zeros((H, 1, C), jnp.float32)
    zrow = jnp.zeros((1, 2, W, 3 * C), jnp.float32)
    for a in range(2):
        z0 = zp[a * 2 + 0]
        z1 = zp[a * 2 + 1]
        cpz_ref[a, 0:1] = zrow
        cpz_ref[a, H + 1:H + 2] = zrow
        cpz_ref[a, 1:H + 1, 0, 0:1, 0:C] = zcol
        cpz_ref[a, 1:H + 1, 0, 1:W, 0:C] = z1[:, 0:W - 1, :]
        cpz_ref[a, 1:H + 1, 0, :, C:2 * C] = z0
        cpz_ref[a, 1:H + 1, 0, :, 2 * C:3 * C] = z1
        cpz_ref[a, 1:H + 1, 1, :, 0:C] = z0
        cpz_ref[a, 1:H + 1, 1, :, C:2 * C] = z1
        cpz_ref[a, 1:H + 1, 1, 0:W - 1, 2 * C:3 * C] = z0[:, 1:W, :]
        cpz_ref[a, 1:H + 1, 1, W - 1:W, 2 * C:3 * C] = zcol

    xs = x_ref[0].reshape(M, C)
    skip = jnp.dot(xs, w3_ref[...],
                   preferred_element_type=jnp.float32) + b3_ref[...]
    skip2 = jnp.broadcast_to(skip.reshape(H, 1, W, Cout),
                             (H, 2, W, Cout)).reshape(2 * M, Cout)

    flats = [cpz_ref[a].reshape((H + 2) * 2 * W, 3 * C) for a in range(2)]
    rows = []
    for ap in range(2):
        acc = None
        for dy in range(3):
            m = ap - 1 + dy
            a_src = m % 2
            u = m // 2
            lhs = flats[a_src][(1 + u) * 2 * W:(1 + u) * 2 * W + 2 * M]
            d = jnp.dot(lhs, w2_ref[dy],
                        preferred_element_type=jnp.float32)
            acc = d if acc is None else acc + d
        outv = acc + b2_ref[...] + skip2
        w = jnp.swapaxes(outv.reshape(H, 2, W, Cout), 1, 2)
        rows.append(w.reshape(H, 2 * W, Cout))
    out_ref[0] = jnp.stack(rows, axis=1).reshape(2 * H, 2 * W, Cout)


def kernel(x, feat, w1, b1, w2, b2, w3, b3, fcw1_w, fcw1_b, fcb1_w, fcb1_b,
           fcw2_w, fcw2_b, fcb2_w, fcb2_b):
    N, Cin, H, W = x.shape
    Cout = w2.shape[0]
    f32, bf16 = jnp.float32, jnp.bfloat16

    x_lo = jnp.transpose(x, (0, 2, 3, 1)).astype(f32)

    whwio = jnp.transpose(w1, (2, 3, 1, 0))
    rowc = [[whwio[0], whwio[1] + whwio[2]],
            [whwio[0] + whwio[1], whwio[2]]]
    blocks = []
    for a in range(2):
        for b in range(2):
            for u in range(2):
                r = rowc[a][u]
                colc = [[r[0], r[1] + r[2]], [r[0] + r[1], r[2]]]
                blocks.append(jnp.concatenate([colc[b][0], colc[b][1]],
                                              axis=0))
    w1p = jnp.stack(blocks).astype(f32)

    w2r = jnp.transpose(w2, (2, 3, 1, 0)).reshape(3, 3 * Cin, Cout).astype(f32)
    w3m = jnp.transpose(w3[:, :, 0, 0], (1, 0)).astype(f32)
    b1m = b1.reshape(1, Cin).astype(f32)
    b2m = b2.reshape(1, Cout).astype(f32)
    b3m = b3.reshape(1, Cout).astype(f32)
    fc_w = jnp.concatenate([fcw1_w, fcb1_w, fcw2_w, fcb2_w],
                           axis=1).astype(f32)
    fc_b = jnp.concatenate([fcw1_b, fcb1_b, fcw2_b, fcb2_b]).reshape(1, 4 * Cin)

    vmem = pl.BlockSpec(memory_space=pltpu.MemorySpace.VMEM)

    s1, sh1, aff2 = pl.pallas_call(
        _prep_kernel,
        out_shape=(jax.ShapeDtypeStruct((N, 1, Cin), f32),
                   jax.ShapeDtypeStruct((N, 1, Cin), f32),
                   jax.ShapeDtypeStruct((N, 1, 2 * Cin), f32)),
        in_specs=[vmem] * 4,
        out_specs=(vmem, vmem, vmem),
        compiler_params=pltpu.CompilerParams(vmem_limit_bytes=VMEM_LIMIT),
    )(x_lo, feat.astype(f32), fc_w, fc_b)

    t, part = pl.pallas_call(
        _conv1_kernel,
        out_shape=(jax.ShapeDtypeStruct((N, 4, H, W, Cin), bf16),
                   jax.ShapeDtypeStruct((N, 2, Cin), f32)),
        grid=(N,),
        in_specs=[pl.BlockSpec((1, H, W, Cin), lambda n: (n, 0, 0, 0)),
                  pl.BlockSpec((1, 1, Cin), lambda n: (n, 0, 0)),
                  pl.BlockSpec((1, 1, Cin), lambda n: (n, 0, 0)),
                  pl.BlockSpec((8, 2 * Cin, Cin), lambda n: (0, 0, 0)),
                  pl.BlockSpec((1, Cin), lambda n: (0, 0))],
        out_specs=(pl.BlockSpec((1, 4, H, W, Cin), lambda n: (n, 0, 0, 0, 0)),
                   pl.BlockSpec((1, 2, Cin), lambda n: (n, 0, 0))),
        scratch_shapes=[pltpu.VMEM((H + 2, W, 3 * Cin), f32)],
        compiler_params=pltpu.CompilerParams(
            dimension_semantics=("parallel",), vmem_limit_bytes=VMEM_LIMIT),
    )(x_lo, s1, sh1, w1p, b1m)

    out_nhwc = pl.pallas_call(
        _conv2_kernel,
        out_shape=jax.ShapeDtypeStruct((N, 2 * H, 2 * W, Cout), f32),
        grid=(N,),
        in_specs=[pl.BlockSpec((1, 4, H, W, Cin), lambda n: (n, 0, 0, 0, 0)),
                  pl.BlockSpec((N, 2, Cin), lambda n: (0, 0, 0)),
                  pl.BlockSpec((1, 1, 2 * Cin), lambda n: (n, 0, 0)),
                  pl.BlockSpec((1, H, W, Cin), lambda n: (n, 0, 0, 0)),
                  pl.BlockSpec((Cin, Cout), lambda n: (0, 0)),
                  pl.BlockSpec((1, Cout), lambda n: (0, 0)),
                  pl.BlockSpec((3, 3 * Cin, Cout), lambda n: (0, 0, 0)),
                  pl.BlockSpec((1, Cout), lambda n: (0, 0))],
        out_specs=pl.BlockSpec((1, 2 * H, 2 * W, Cout),
                               lambda n: (n, 0, 0, 0)),
        scratch_shapes=[pltpu.VMEM((2, H + 2, 2, W, 3 * Cin), f32)],
        compiler_params=pltpu.CompilerParams(
            dimension_semantics=("parallel",), vmem_limit_bytes=VMEM_LIMIT),
    )(t, part, aff2, x_lo, w3m, b3m, w2r, b2m)

    return jnp.transpose(out_nhwc, (0, 3, 1, 2))


# speedup vs baseline: 1.9427x; 1.0080x over previous
import jax
import jax.numpy as jnp
from jax.experimental import pallas as pl
from jax.experimental.pallas import tpu as pltpu

EPS = 1e-5
VMEM_LIMIT = 32 * 1024 * 1024


def _build_colpat(cp_ref, img):
    H, W, C = img.shape
    z = jnp.zeros((1, W, 3 * C), jnp.float32)
    cp_ref[0:1] = z
    cp_ref[H + 1:H + 2] = z
    zc = jnp.zeros((H, 1, C), jnp.float32)
    cp_ref[1:H + 1, 0:1, 0:C] = zc
    cp_ref[1:H + 1, 1:W, 0:C] = img[:, 0:W - 1, :]
    cp_ref[1:H + 1, :, C:2 * C] = img
    cp_ref[1:H + 1, 0:W - 1, 2 * C:3 * C] = img[:, 1:W, :]
    cp_ref[1:H + 1, W - 1:W, 2 * C:3 * C] = zc


def _prep_kernel(x_ref, feat_ref, fcw_ref, fcb_ref, s1_ref, sh1_ref, aff2_ref):
    N, H, W, C = x_ref.shape
    M = N * H * W
    x = x_ref[...].reshape(M, C)
    f = feat_ref[...]
    wv1 = jnp.dot(f, fcw_ref[0],
                  preferred_element_type=jnp.float32) + fcb_ref[:, 0 * C:1 * C]
    bv1 = jnp.dot(f, fcw_ref[1],
                  preferred_element_type=jnp.float32) + fcb_ref[:, 1 * C:2 * C]
    wv2 = jnp.dot(f, fcw_ref[2],
                  preferred_element_type=jnp.float32) + fcb_ref[:, 2 * C:3 * C]
    bv2 = jnp.dot(f, fcw_ref[3],
                  preferred_element_type=jnp.float32) + fcb_ref[:, 3 * C:4 * C]
    mean1 = jnp.sum(x, axis=0, keepdims=True) / M
    ex2 = jnp.sum(x * x, axis=0, keepdims=True) / M
    inv1 = jax.lax.rsqrt(ex2 - mean1 * mean1 + EPS)
    s1_ref[...] = (wv1 * inv1).reshape(N, 1, C)
    sh1_ref[...] = (bv1 - wv1 * mean1 * inv1).reshape(N, 1, C)
    aff2_ref[...] = jnp.concatenate([wv2, bv2], axis=1).reshape(N, 1, 2 * C)


def _conv1_kernel(x_ref, s1_ref, sh1_ref, w1p_ref, b1_ref, t_ref, part_ref,
                  cp_ref):
    _, H, W, C = x_ref.shape
    M = H * W
    x = x_ref[0].reshape(M, C)
    h = jnp.maximum(s1_ref[0] * x + sh1_ref[0], 0.0)
    _build_colpat(cp_ref, h.reshape(H, W, C))
    flat = cp_ref[...].reshape((H + 2) * W, 3 * C)
    s = None
    s2 = None
    for a in range(2):
        for b in range(2):
            acc = None
            for u in range(2):
                lhs = flat[(a + u) * W:(a + u) * W + M, b * C:(b + 2) * C]
                d = jnp.dot(lhs, w1p_ref[(a * 2 + b) * 2 + u],
                            preferred_element_type=jnp.float32)
                acc = d if acc is None else acc + d
            conv = acc + b1_ref[...]
            t_ref[0, a * 2 + b] = conv.astype(jnp.bfloat16).reshape(H, W, C)
            ps = jnp.sum(conv, axis=0, keepdims=True)
            ps2 = jnp.sum(conv * conv, axis=0, keepdims=True)
            s = ps if s is None else s + ps
            s2 = ps2 if s2 is None else s2 + ps2
    part_ref[0, 0:1, :] = s
    part_ref[0, 1:2, :] = s2


def _conv2_kernel(t_ref, part_ref, aff2_ref, x_ref, w3_ref, b3_ref, w2_ref,
                  b2_ref, out_ref, cpz_ref):
    _, _, H, W, C = t_ref.shape
    M = H * W
    N = part_ref.shape[0]
    Cout = w3_ref.shape[1]
    M2 = N * 4 * H * W
    mean2 = jnp.sum(part_ref[:, 0, :], axis=0, keepdims=True) / M2
    ex2 = jnp.sum(part_ref[:, 1, :], axis=0, keepdims=True) / M2
    inv2 = jax.lax.rsqrt(ex2 - mean2 * mean2 + EPS)
    wv2 = aff2_ref[0, :, 0:C]
    bv2 = aff2_ref[0, :, C:2 * C]
    sc = wv2 * inv2
    sh = bv2 - wv2 * mean2 * inv2

    zp = []
    for ph in range(4):
        tp = t_ref[0, ph].reshape(M, C).astype(jnp.float32)
        zp.append(jnp.maximum(sc * tp + sh, 0.0).reshape(H, W, C))

    zcol = jnp.zeros((H, 1, C), jnp.float32)
    zrow = jnp.zeros((1, 2, W, 3 * C), jnp.float32)
    for a in range(2):
        z0 = zp[a * 2 + 0]
        z1 = zp[a * 2 + 1]
        cpz_ref[a, 0:1] = zrow
        cpz_ref[a, H + 1:H + 2] = zrow
        cpz_ref[a, 1:H + 1, 0, 0:1, 0:C] = zcol
        cpz_ref[a, 1:H + 1, 0, 1:W, 0:C] = z1[:, 0:W - 1, :]
        cpz_ref[a, 1:H + 1, 0, :, C:2 * C] = z0
        cpz_ref[a, 1:H + 1, 0, :, 2 * C:3 * C] = z1
        cpz_ref[a, 1:H + 1, 1, :, 0:C] = z0
        cpz_ref[a, 1:H + 1, 1, :, C:2 * C] = z1
        cpz_ref[a, 1:H + 1, 1, 0:W - 1, 2 * C:3 * C] = z0[:, 1:W, :]
        cpz_ref[a, 1:H + 1, 1, W - 1:W, 2 * C:3 * C] = zcol

    xs = x_ref[0].reshape(M, C)
    skip = jnp.dot(xs, w3_ref[...],
                   preferred_element_type=jnp.float32) + b3_ref[...]
    skip2 = jnp.broadcast_to(skip.reshape(H, 1, W, Cout),
                             (H, 2, W, Cout)).reshape(2 * M, Cout)

    flats = [cpz_ref[a].reshape((H + 2) * 2 * W, 3 * C) for a in range(2)]
    rows = []
    for ap in range(2):
        acc = None
        for dy in range(3):
            m = ap - 1 + dy
            a_src = m % 2
            u = m // 2
            lhs = flats[a_src][(1 + u) * 2 * W:(1 + u) * 2 * W + 2 * M]
            d = jnp.dot(lhs, w2_ref[dy],
                        preferred_element_type=jnp.float32)
            acc = d if acc is None else acc + d
        outv = acc + b2_ref[...] + skip2
        w = jnp.swapaxes(outv.reshape(H, 2, W, Cout), 1, 2)
        rows.append(w.reshape(H, 2 * W, Cout))
    out_ref[0] = jnp.stack(rows, axis=1).reshape(2 * H, 2 * W, Cout)


def kernel(x, feat, w1, b1, w2, b2, w3, b3, fcw1_w, fcw1_b, fcb1_w, fcb1_b,
           fcw2_w, fcw2_b, fcb2_w, fcb2_b):
    N, Cin, H, W = x.shape
    Cout = w2.shape[0]
    f32, bf16 = jnp.float32, jnp.bfloat16

    x_lo = jnp.transpose(x, (0, 2, 3, 1)).astype(f32)

    whwio = jnp.transpose(w1, (2, 3, 1, 0))
    rowc = [[whwio[0], whwio[1] + whwio[2]],
            [whwio[0] + whwio[1], whwio[2]]]
    blocks = []
    for a in range(2):
        for b in range(2):
            for u in range(2):
                r = rowc[a][u]
                colc = [[r[0], r[1] + r[2]], [r[0] + r[1], r[2]]]
                blocks.append(jnp.concatenate([colc[b][0], colc[b][1]],
                                              axis=0))
    w1p = jnp.stack(blocks).astype(f32)

    w2r = jnp.transpose(w2, (2, 3, 1, 0)).reshape(3, 3 * Cin, Cout).astype(f32)
    w3m = jnp.transpose(w3[:, :, 0, 0], (1, 0)).astype(f32)
    b1m = b1.reshape(1, Cin).astype(f32)
    b2m = b2.reshape(1, Cout).astype(f32)
    b3m = b3.reshape(1, Cout).astype(f32)
    fc_w = jnp.stack([fcw1_w, fcb1_w, fcw2_w, fcb2_w]).astype(f32)
    fc_b = jnp.concatenate([fcw1_b, fcb1_b, fcw2_b, fcb2_b]).reshape(1, 4 * Cin)

    vmem = pl.BlockSpec(memory_space=pltpu.MemorySpace.VMEM)

    s1, sh1, aff2 = pl.pallas_call(
        _prep_kernel,
        out_shape=(jax.ShapeDtypeStruct((N, 1, Cin), f32),
                   jax.ShapeDtypeStruct((N, 1, Cin), f32),
                   jax.ShapeDtypeStruct((N, 1, 2 * Cin), f32)),
        in_specs=[vmem] * 4,
        out_specs=(vmem, vmem, vmem),
        compiler_params=pltpu.CompilerParams(vmem_limit_bytes=VMEM_LIMIT),
    )(x_lo, feat.astype(f32), fc_w, fc_b)

    t, part = pl.pallas_call(
        _conv1_kernel,
        out_shape=(jax.ShapeDtypeStruct((N, 4, H, W, Cin), bf16),
                   jax.ShapeDtypeStruct((N, 2, Cin), f32)),
        grid=(N,),
        in_specs=[pl.BlockSpec((1, H, W, Cin), lambda n: (n, 0, 0, 0)),
                  pl.BlockSpec((1, 1, Cin), lambda n: (n, 0, 0)),
                  pl.BlockSpec((1, 1, Cin), lambda n: (n, 0, 0)),
                  pl.BlockSpec((8, 2 * Cin, Cin), lambda n: (0, 0, 0)),
                  pl.BlockSpec((1, Cin), lambda n: (0, 0))],
        out_specs=(pl.BlockSpec((1, 4, H, W, Cin), lambda n: (n, 0, 0, 0, 0)),
                   pl.BlockSpec((1, 2, Cin), lambda n: (n, 0, 0))),
        scratch_shapes=[pltpu.VMEM((H + 2, W, 3 * Cin), f32)],
        compiler_params=pltpu.CompilerParams(
            dimension_semantics=("parallel",), vmem_limit_bytes=VMEM_LIMIT),
    )(x_lo, s1, sh1, w1p, b1m)

    out_nhwc = pl.pallas_call(
        _conv2_kernel,
        out_shape=jax.ShapeDtypeStruct((N, 2 * H, 2 * W, Cout), f32),
        grid=(N,),
        in_specs=[pl.BlockSpec((1, 4, H, W, Cin), lambda n: (n, 0, 0, 0, 0)),
                  pl.BlockSpec((N, 2, Cin), lambda n: (0, 0, 0)),
                  pl.BlockSpec((1, 1, 2 * Cin), lambda n: (n, 0, 0)),
                  pl.BlockSpec((1, H, W, Cin), lambda n: (n, 0, 0, 0)),
                  pl.BlockSpec((Cin, Cout), lambda n: (0, 0)),
                  pl.BlockSpec((1, Cout), lambda n: (0, 0)),
                  pl.BlockSpec((3, 3 * Cin, Cout), lambda n: (0, 0, 0)),
                  pl.BlockSpec((1, Cout), lambda n: (0, 0))],
        out_specs=pl.BlockSpec((1, 2 * H, 2 * W, Cout),
                               lambda n: (n, 0, 0, 0)),
        scratch_shapes=[pltpu.VMEM((2, H + 2, 2, W, 3 * Cin), f32)],
        compiler_params=pltpu.CompilerParams(
            dimension_semantics=("parallel",), vmem_limit_bytes=VMEM_LIMIT),
    )(t, part, aff2, x_lo, w3m, b3m, w2r, b2m)

    return jnp.transpose(out_nhwc, (0, 3, 1, 2))
